# baseline (speedup 1.0000x reference)
import jax
import jax.numpy as jnp
from jax import lax
from jax.experimental import pallas as pl
from jax.experimental.pallas import tpu as pltpu
from jax.experimental.pallas import tpu_sc as plsc

R = 128
V = 100000
NC, NS = 2, 16
NW = NC * NS
RPW = R // NW
C = 10000
NCHUNK = V // C
BLK = 400
BPC = C // BLK
NBLK = V // BLK
M = BLK // 16
SUP = 5
DEPTH = 4
NSLAB = RPW * NCHUNK
TOPK = 16

_CONST_CACHE = None


def _consts():
    global _CONST_CACHE
    if _CONST_CACHE is None:
        with jax.ensure_compile_time_eval():
            ekey = jax.random.key(42)
            e = jax.random.exponential(ekey, (R, V), dtype=jnp.float32)
            etab = jnp.log(jnp.clip(e, 1e-10, None))
            neg = (-etab).reshape(R, NBLK, M, 16)
            nmax = jnp.max(neg, axis=2).reshape(R, NBLK * 16)
            nmin = jnp.min(neg, axis=2).reshape(R, NBLK * 16)
            tvals, tidx = lax.top_k(-etab, TOPK)
            _CONST_CACHE = (etab, nmax, nmin, tidx.astype(jnp.int32), -tvals)
    return _CONST_CACHE


def _body(logits_hbm, e_hbm, ts_hbm, em_hbm, mlb_hbm, nmax_hbm, nmin_hbm,
          out_hbm, lbufs, xbufs, nbufs, svl, sve, svid,
          tbuf, embuf, mbuf, obuf, bmax, sls, sxs, sns, svsem):
    cid = lax.axis_index("c")
    sid = lax.axis_index("s")
    wid = cid * NS + sid
    base = wid * RPW

    pltpu.sync_copy(ts_hbm.at[pl.ds(base * 16, RPW * 16)], tbuf)
    pltpu.sync_copy(em_hbm.at[pl.ds(base * 16, RPW * 16)], embuf)
    pltpu.sync_copy(mlb_hbm.at[pl.ds(base * 16, RPW * 16)], mbuf)

    iota = lax.iota(jnp.int32, 16)
    big = jnp.full((16,), jnp.int32(2147483647), jnp.int32)
    ninf = jnp.full((16,), -jnp.inf, jnp.float32)
    zero = jnp.zeros((16,), jnp.int32)
    one = jnp.int32(1)

    def start(r, k, b):
        row = base + r
        pltpu.async_copy(logits_hbm.at[row, pl.ds(k * C, C)], lbufs[b], sls[b])
        pltpu.async_copy(
            nmax_hbm.at[row, pl.ds(k * BPC * 16, BPC * 16)], xbufs[b], sxs[b])
        pltpu.async_copy(
            nmin_hbm.at[row, pl.ds(k * BPC * 16, BPC * 16)], nbufs[b], sns[b])

    def wait(b):
        pltpu.make_async_copy(
            logits_hbm.at[base, pl.ds(0, C)], lbufs[b], sls[b]).wait()
        pltpu.make_async_copy(
            nmax_hbm.at[base, pl.ds(0, BPC * 16)], xbufs[b], sxs[b]).wait()
        pltpu.make_async_copy(
            nmin_hbm.at[base, pl.ds(0, BPC * 16)], nbufs[b], sns[b]).wait()

    def roll(r, k, n=1):
        for _ in range(n):
            nk = k + 1
            wrapped = nk == NCHUNK
            r = r + jnp.where(wrapped, 1, 0).astype(jnp.int32)
            k = jnp.where(wrapped, 0, nk).astype(jnp.int32)
        return r, k

    def eval_prev(ring, nsP, kP, tvP, emvP, vm, vi):
        def do(args):
            vm, vi = args

            def ev(i, c2):
                vm, vi = c2
                pltpu.make_async_copy(
                    logits_hbm.at[base, pl.ds(0, BLK)],
                    svl[ring].at[pl.ds(i * BLK, BLK)], svsem[ring]).wait()
                pltpu.make_async_copy(
                    logits_hbm.at[base, pl.ds(0, BLK)],
                    sve[ring].at[pl.ds(i * BLK, BLK)], svsem[ring]).wait()
                bidvec = svid[ring][pl.ds(i * 16, 16)]

                def ev1(ii, c3):
                    vm, vi = c3
                    off = i * BLK + ii * 16
                    l = svl[ring][pl.ds(off, 16)]
                    e = sve[ring][pl.ds(off, 16)]
                    v = l / tvP - emvP * e
                    cur = bidvec * M + jnp.full((16,), ii, jnp.int32) \
                        + jnp.broadcast_to(kP * (BPC * M), (16,))
                    mk = v > vm
                    return jnp.where(mk, v, vm), jnp.where(mk, cur, vi)

                return lax.fori_loop(0, M, ev1, (vm, vi))

            return lax.fori_loop(0, nsP, ev, (vm, vi))

        return lax.cond(nsP > 0, do, lambda a: a, (vm, vi))

    def phase_a(ring, k, row, tv, emv, mlb_vec, lref, xref, nref):
        mlb_scalar = jnp.max(mlb_vec)

        def bm(j, carry):
            b0 = j * BLK
            chains = [None] * 5
            for i in range(M):
                x = lref[pl.ds(b0 + i * 16, 16)]
                cs = chains[i % 5]
                chains[i % 5] = x if cs is None else jnp.maximum(cs, x)
            lanemax = jnp.maximum(
                jnp.maximum(jnp.maximum(chains[0], chains[1]),
                            jnp.maximum(chains[2], chains[3])), chains[4])
            bmax[pl.ds(j * 16, 16)] = lanemax
            return carry

        lax.fori_loop(0, BPC, bm, 0)

        def bd(j, carry):
            mlb_vec, bitsvec = carry
            lm = bmax[pl.ds(j * 16, 16)]
            a = lm / tv
            nx = xref[pl.ds(j * 16, 16)]
            nn = nref[pl.ds(j * 16, 16)]
            ub = a + emv * nx
            lb = a + emv * nn
            mlb_vec = jnp.maximum(mlb_vec, lb)
            bitj = jnp.broadcast_to(one << j, (16,))
            bitsvec = bitsvec | jnp.where(ub >= mlb_scalar, bitj, zero)
            return mlb_vec, bitsvec

        mlb_vec, bitsvec = lax.fori_loop(0, BPC, bd, (mlb_vec, zero))

        pc_any = plsc.all_reduce_population_count(bitsvec != 0)

        def issue_all(_):
            def iss(j, ns):
                mj = (bitsvec & jnp.broadcast_to(one << j, (16,))) != 0
                pcj = plsc.all_reduce_population_count(mj)
                hit = pcj[0] > 0

                @pl.when(hit)
                def _():
                    pltpu.async_copy(
                        logits_hbm.at[row, pl.ds(k * C + j * BLK, BLK)],
                        svl[ring].at[pl.ds(ns * BLK, BLK)], svsem[ring])
                    pltpu.async_copy(
                        e_hbm.at[row, pl.ds(k * C + j * BLK, BLK)],
                        sve[ring].at[pl.ds(ns * BLK, BLK)], svsem[ring])
                    svid[ring][pl.ds(ns * 16, 16)] = jnp.broadcast_to(j, (16,)).astype(jnp.int32)

                return ns + jnp.where(hit, 1, 0).astype(jnp.int32)

            return lax.fori_loop(0, BPC, iss, jnp.int32(0))

        nsurv = lax.cond(pc_any[0] > 0, issue_all, lambda _: jnp.int32(0), 0)
        return mlb_vec, nsurv

    for p in range(DEPTH - 1):
        start(p // NCHUNK, p % NCHUNK, p)

    def slab(p, st):
        (r, k, rs, ks, vm, vi, mlb_vec, ovec,
         nsP, kP, rP, tvP, emvP) = st
        ring = p % 2
        pring = (p + 1) % 2
        wait(p)

        @pl.when(rs < RPW)
        def _():
            start(rs, ks, (p + DEPTH - 1) % DEPTH)

        vm, vi = eval_prev(pring, nsP, kP, tvP, emvP, vm, vi)

        m_all = jnp.max(vm)
        cand = jnp.where(vm == m_all, vi * 16 + iota, big)
        best = jnp.min(cand)
        fin = (kP == NCHUNK - 1) & (nsP >= 0)
        ovec = jnp.where((iota == rP) & fin, best, ovec)

        fresh = k == 0
        vm = jnp.where(fresh, ninf, vm)
        vi = jnp.where(fresh, zero, vi)
        mlb_vec = jnp.where(fresh, mbuf[pl.ds(r * 16, 16)], mlb_vec)

        tv = tbuf[pl.ds(r * 16, 16)]
        emv = embuf[pl.ds(r * 16, 16)]
        mlb_vec, nsurv = phase_a(ring, k, base + r, tv, emv, mlb_vec,
                                 lbufs[p], xbufs[p], nbufs[p])

        nsP, kP, rP, tvP, emvP = nsurv, k, r, tv, emv
        r, k = roll(r, k)
        rs, ks = roll(rs, ks)
        return (r, k, rs, ks, vm, vi, mlb_vec, ovec, nsP, kP, rP, tvP, emvP)

    st = (jnp.int32(0), jnp.int32(0),
          jnp.int32((DEPTH - 1) // NCHUNK), jnp.int32((DEPTH - 1) % NCHUNK),
          ninf, zero, ninf, zero,
          jnp.int32(0), jnp.int32(-1), jnp.int32(0),
          ninf, ninf)

    def iter4(i, st):
        for p in range(DEPTH):
            st = slab(p, st)
        return st

    st = lax.fori_loop(0, NSLAB // DEPTH, iter4, st)

    (_, _, _, _, vm, vi, _, ovec, nsP, kP, rP, tvP, emvP) = st
    vm, vi = eval_prev((NSLAB - 1) % 2, nsP, kP, tvP, emvP, vm, vi)
    m_all = jnp.max(vm)
    cand = jnp.where(vm == m_all, vi * 16 + iota, big)
    best = jnp.min(cand)
    ovec = jnp.where(iota == rP, best, ovec)

    obuf[...] = ovec
    pltpu.sync_copy(obuf, out_hbm.at[wid])


@jax.jit
def _sample(logits, temps, etab, nmax, nmin, tidx, tval):
    greedy = temps == 0.0
    ts = jnp.where(greedy, 1.0, temps).astype(jnp.float32)
    em = jnp.where(greedy, 0.0, 1.0).astype(jnp.float32)
    ts_b = jnp.broadcast_to(ts[:, None], (R, 16)).reshape(-1)
    em_b = jnp.broadcast_to(em[:, None], (R, 16)).reshape(-1)

    lt = jnp.take_along_axis(logits, tidx, axis=1)
    vtop = jnp.where(greedy[:, None], lt, lt / ts[:, None] - tval)
    mlb0 = jnp.max(vtop, axis=1)
    mlb_b = jnp.broadcast_to(mlb0[:, None], (R, 16)).reshape(-1)

    mesh = plsc.VectorSubcoreMesh(
        core_axis_name="c", subcore_axis_name="s", num_cores=NC, num_subcores=NS
    )
    run = pl.kernel(
        _body,
        out_type=jax.ShapeDtypeStruct((NW, 16), jnp.int32),
        mesh=mesh,
        compiler_params=pltpu.CompilerParams(
            use_tc_tiling_on_sc=False, needs_layout_passes=False
        ),
        scratch_types=[
            [pltpu.VMEM((C,), jnp.float32) for _ in range(DEPTH)],
            [pltpu.VMEM((BPC * 16,), jnp.float32) for _ in range(DEPTH)],
            [pltpu.VMEM((BPC * 16,), jnp.float32) for _ in range(DEPTH)],
            [pltpu.VMEM((BPC * BLK,), jnp.float32) for _ in range(2)],
            [pltpu.VMEM((BPC * BLK,), jnp.float32) for _ in range(2)],
            [pltpu.VMEM((BPC * 16,), jnp.int32) for _ in range(2)],
            pltpu.VMEM((RPW * 16,), jnp.float32),
            pltpu.VMEM((RPW * 16,), jnp.float32),
            pltpu.VMEM((RPW * 16,), jnp.float32),
            pltpu.VMEM((16,), jnp.int32),
            pltpu.VMEM((BPC * 16,), jnp.float32),
            [pltpu.SemaphoreType.DMA for _ in range(DEPTH)],
            [pltpu.SemaphoreType.DMA for _ in range(DEPTH)],
            [pltpu.SemaphoreType.DMA for _ in range(DEPTH)],
            [pltpu.SemaphoreType.DMA for _ in range(2)],
        ],
    )
    res = run(logits, etab, ts_b, em_b, mlb_b, nmax, nmin)
    return res[:, :RPW].reshape(-1)


def kernel(logits, temperatures):
    etab, nmax, nmin, tidx, tval = _consts()
    temps = temperatures.reshape(-1).astype(jnp.float32)
    return _sample(logits.astype(jnp.float32), temps, etab, nmax, nmin,
                   tidx, tval)


# speedup vs baseline: 1.0186x; 1.0186x over previous
import jax
import jax.numpy as jnp
from jax import lax
from jax.experimental import pallas as pl
from jax.experimental.pallas import tpu as pltpu
from jax.experimental.pallas import tpu_sc as plsc

R = 128
V = 100000
NC, NS = 2, 16
NW = NC * NS
RPW = R // NW
C = 10000
NCHUNK = V // C
BLK = 400
BPC = C // BLK
NBLK = V // BLK
M = BLK // 16
SUP = 5
DEPTH = 4
NSLAB = RPW * NCHUNK
TOPK = 16

_CONST_CACHE = None


def _consts():
    global _CONST_CACHE
    if _CONST_CACHE is None:
        with jax.ensure_compile_time_eval():
            ekey = jax.random.key(42)
            e = jax.random.exponential(ekey, (R, V), dtype=jnp.float32)
            etab = jnp.log(jnp.clip(e, 1e-10, None))
            neg = (-etab).reshape(R, NBLK, M, 16)
            nmax = jnp.max(neg, axis=2).reshape(R, NBLK * 16)
            nmin = jnp.min(neg, axis=2).reshape(R, NBLK * 16)
            tvals, tidx = lax.top_k(-etab, TOPK)
            _CONST_CACHE = (etab, nmax, nmin, tidx.astype(jnp.int32), -tvals)
    return _CONST_CACHE


def _body(logits_hbm, e_hbm, ts_hbm, em_hbm, mlb_hbm, nmax_hbm, nmin_hbm,
          out_hbm, lbufs, xbufs, nbufs, svl, sve, svid,
          tbuf, embuf, mbuf, obuf, bmax, sls, sxs, sns, svsem):
    cid = lax.axis_index("c")
    sid = lax.axis_index("s")
    wid = cid * NS + sid
    base = wid * RPW

    pltpu.sync_copy(ts_hbm.at[pl.ds(base * 16, RPW * 16)], tbuf)
    pltpu.sync_copy(em_hbm.at[pl.ds(base * 16, RPW * 16)], embuf)
    pltpu.sync_copy(mlb_hbm.at[pl.ds(base * 16, RPW * 16)], mbuf)

    iota = lax.iota(jnp.int32, 16)
    big = jnp.full((16,), jnp.int32(2147483647), jnp.int32)
    ninf = jnp.full((16,), -jnp.inf, jnp.float32)
    zero = jnp.zeros((16,), jnp.int32)
    one = jnp.int32(1)

    def start(r, k, b):
        row = base + r
        pltpu.async_copy(logits_hbm.at[row, pl.ds(k * C, C)], lbufs[b], sls[b])
        pltpu.async_copy(
            nmax_hbm.at[row, pl.ds(k * BPC * 16, BPC * 16)], xbufs[b], sxs[b])
        pltpu.async_copy(
            nmin_hbm.at[row, pl.ds(k * BPC * 16, BPC * 16)], nbufs[b], sns[b])

    def wait(b):
        pltpu.make_async_copy(
            logits_hbm.at[base, pl.ds(0, C)], lbufs[b], sls[b]).wait()
        pltpu.make_async_copy(
            nmax_hbm.at[base, pl.ds(0, BPC * 16)], xbufs[b], sxs[b]).wait()
        pltpu.make_async_copy(
            nmin_hbm.at[base, pl.ds(0, BPC * 16)], nbufs[b], sns[b]).wait()

    def roll(r, k, n=1):
        for _ in range(n):
            nk = k + 1
            wrapped = nk == NCHUNK
            r = r + jnp.where(wrapped, 1, 0).astype(jnp.int32)
            k = jnp.where(wrapped, 0, nk).astype(jnp.int32)
        return r, k

    def eval_prev(ring, nsP, kP, tvP, emvP, vm, vi):
        def do(args):
            vm, vi = args

            def ev(i, c2):
                vm, vi = c2
                pltpu.make_async_copy(
                    logits_hbm.at[base, pl.ds(0, BLK)],
                    svl[ring].at[pl.ds(i * BLK, BLK)], svsem[ring]).wait()
                pltpu.make_async_copy(
                    logits_hbm.at[base, pl.ds(0, BLK)],
                    sve[ring].at[pl.ds(i * BLK, BLK)], svsem[ring]).wait()
                bidvec = svid[ring][pl.ds(i * 16, 16)]

                def ev1(ii, c3):
                    vm, vi = c3
                    off = i * BLK + ii * 16
                    l = svl[ring][pl.ds(off, 16)]
                    e = sve[ring][pl.ds(off, 16)]
                    v = l / tvP - emvP * e
                    cur = bidvec * M + jnp.full((16,), ii, jnp.int32) \
                        + jnp.broadcast_to(kP * (BPC * M), (16,))
                    mk = v > vm
                    return jnp.where(mk, v, vm), jnp.where(mk, cur, vi)

                return lax.fori_loop(0, M, ev1, (vm, vi))

            return lax.fori_loop(0, nsP, ev, (vm, vi))

        return lax.cond(nsP > 0, do, lambda a: a, (vm, vi))

    def phase_a(ring, k, row, tv, emv, mlb_vec, lref, xref, nref):
        mlb_scalar = jnp.max(mlb_vec)

        @plsc.parallel_loop(0, BPC, unroll=2)
        def bm(j):
            b0 = j * BLK
            chains = [None] * 5
            for i in range(M):
                x = lref[pl.ds(b0 + i * 16, 16)]
                cs = chains[i % 5]
                chains[i % 5] = x if cs is None else jnp.maximum(cs, x)
            lanemax = jnp.maximum(
                jnp.maximum(jnp.maximum(chains[0], chains[1]),
                            jnp.maximum(chains[2], chains[3])), chains[4])
            bmax[pl.ds(j * 16, 16)] = lanemax

        def bd(j, carry):
            mlb_vec, bitsvec = carry
            lm = bmax[pl.ds(j * 16, 16)]
            a = lm / tv
            nx = xref[pl.ds(j * 16, 16)]
            nn = nref[pl.ds(j * 16, 16)]
            ub = a + emv * nx
            lb = a + emv * nn
            mlb_vec = jnp.maximum(mlb_vec, lb)
            bitj = jnp.broadcast_to(one << j, (16,))
            bitsvec = bitsvec | jnp.where(ub >= mlb_scalar, bitj, zero)
            return mlb_vec, bitsvec

        mlb_vec, bitsvec = plsc.parallel_loop(
            0, BPC, unroll=5, carry=(mlb_vec, zero))(bd)

        pc_any = plsc.all_reduce_population_count(bitsvec != 0)

        def issue_all(_):
            def iss(j, ns):
                mj = (bitsvec & jnp.broadcast_to(one << j, (16,))) != 0
                pcj = plsc.all_reduce_population_count(mj)
                hit = pcj[0] > 0

                @pl.when(hit)
                def _():
                    pltpu.async_copy(
                        logits_hbm.at[row, pl.ds(k * C + j * BLK, BLK)],
                        svl[ring].at[pl.ds(ns * BLK, BLK)], svsem[ring])
                    pltpu.async_copy(
                        e_hbm.at[row, pl.ds(k * C + j * BLK, BLK)],
                        sve[ring].at[pl.ds(ns * BLK, BLK)], svsem[ring])
                    svid[ring][pl.ds(ns * 16, 16)] = jnp.broadcast_to(j, (16,)).astype(jnp.int32)

                return ns + jnp.where(hit, 1, 0).astype(jnp.int32)

            return lax.fori_loop(0, BPC, iss, jnp.int32(0))

        nsurv = lax.cond(pc_any[0] > 0, issue_all, lambda _: jnp.int32(0), 0)
        return mlb_vec, nsurv

    for p in range(DEPTH - 1):
        start(p // NCHUNK, p % NCHUNK, p)

    def slab(p, st):
        (r, k, rs, ks, vm, vi, mlb_vec, ovec,
         nsP, kP, rP, tvP, emvP) = st
        ring = p % 2
        pring = (p + 1) % 2
        wait(p)

        @pl.when(rs < RPW)
        def _():
            start(rs, ks, (p + DEPTH - 1) % DEPTH)

        vm, vi = eval_prev(pring, nsP, kP, tvP, emvP, vm, vi)

        m_all = jnp.max(vm)
        cand = jnp.where(vm == m_all, vi * 16 + iota, big)
        best = jnp.min(cand)
        fin = (kP == NCHUNK - 1) & (nsP >= 0)
        ovec = jnp.where((iota == rP) & fin, best, ovec)

        fresh = k == 0
        vm = jnp.where(fresh, ninf, vm)
        vi = jnp.where(fresh, zero, vi)
        mlb_vec = jnp.where(fresh, mbuf[pl.ds(r * 16, 16)], mlb_vec)

        tv = tbuf[pl.ds(r * 16, 16)]
        emv = embuf[pl.ds(r * 16, 16)]
        mlb_vec, nsurv = phase_a(ring, k, base + r, tv, emv, mlb_vec,
                                 lbufs[p], xbufs[p], nbufs[p])

        nsP, kP, rP, tvP, emvP = nsurv, k, r, tv, emv
        r, k = roll(r, k)
        rs, ks = roll(rs, ks)
        return (r, k, rs, ks, vm, vi, mlb_vec, ovec, nsP, kP, rP, tvP, emvP)

    st = (jnp.int32(0), jnp.int32(0),
          jnp.int32((DEPTH - 1) // NCHUNK), jnp.int32((DEPTH - 1) % NCHUNK),
          ninf, zero, ninf, zero,
          jnp.int32(0), jnp.int32(-1), jnp.int32(0),
          ninf, ninf)

    def iter4(i, st):
        for p in range(DEPTH):
            st = slab(p, st)
        return st

    st = lax.fori_loop(0, NSLAB // DEPTH, iter4, st)

    (_, _, _, _, vm, vi, _, ovec, nsP, kP, rP, tvP, emvP) = st
    vm, vi = eval_prev((NSLAB - 1) % 2, nsP, kP, tvP, emvP, vm, vi)
    m_all = jnp.max(vm)
    cand = jnp.where(vm == m_all, vi * 16 + iota, big)
    best = jnp.min(cand)
    ovec = jnp.where(iota == rP, best, ovec)

    obuf[...] = ovec
    pltpu.sync_copy(obuf, out_hbm.at[wid])


@jax.jit
def _sample(logits, temps, etab, nmax, nmin, tidx, tval):
    greedy = temps == 0.0
    ts = jnp.where(greedy, 1.0, temps).astype(jnp.float32)
    em = jnp.where(greedy, 0.0, 1.0).astype(jnp.float32)
    ts_b = jnp.broadcast_to(ts[:, None], (R, 16)).reshape(-1)
    em_b = jnp.broadcast_to(em[:, None], (R, 16)).reshape(-1)

    lt = jnp.take_along_axis(logits, tidx, axis=1)
    vtop = jnp.where(greedy[:, None], lt, lt / ts[:, None] - tval)
    mlb0 = jnp.max(vtop, axis=1)
    mlb_b = jnp.broadcast_to(mlb0[:, None], (R, 16)).reshape(-1)

    mesh = plsc.VectorSubcoreMesh(
        core_axis_name="c", subcore_axis_name="s", num_cores=NC, num_subcores=NS
    )
    run = pl.kernel(
        _body,
        out_type=jax.ShapeDtypeStruct((NW, 16), jnp.int32),
        mesh=mesh,
        compiler_params=pltpu.CompilerParams(
            use_tc_tiling_on_sc=False, needs_layout_passes=False
        ),
        scratch_types=[
            [pltpu.VMEM((C,), jnp.float32) for _ in range(DEPTH)],
            [pltpu.VMEM((BPC * 16,), jnp.float32) for _ in range(DEPTH)],
            [pltpu.VMEM((BPC * 16,), jnp.float32) for _ in range(DEPTH)],
            [pltpu.VMEM((BPC * BLK,), jnp.float32) for _ in range(2)],
            [pltpu.VMEM((BPC * BLK,), jnp.float32) for _ in range(2)],
            [pltpu.VMEM((BPC * 16,), jnp.int32) for _ in range(2)],
            pltpu.VMEM((RPW * 16,), jnp.float32),
            pltpu.VMEM((RPW * 16,), jnp.float32),
            pltpu.VMEM((RPW * 16,), jnp.float32),
            pltpu.VMEM((16,), jnp.int32),
            pltpu.VMEM((BPC * 16,), jnp.float32),
            [pltpu.SemaphoreType.DMA for _ in range(DEPTH)],
            [pltpu.SemaphoreType.DMA for _ in range(DEPTH)],
            [pltpu.SemaphoreType.DMA for _ in range(DEPTH)],
            [pltpu.SemaphoreType.DMA for _ in range(2)],
        ],
    )
    res = run(logits, etab, ts_b, em_b, mlb_b, nmax, nmin)
    return res[:, :RPW].reshape(-1)


def kernel(logits, temperatures):
    etab, nmax, nmin, tidx, tval = _consts()
    temps = temperatures.reshape(-1).astype(jnp.float32)
    return _sample(logits.astype(jnp.float32), temps, etab, nmax, nmin,
                   tidx, tval)


# speedup vs baseline: 1.2547x; 1.2318x over previous
import functools

import jax
import jax.numpy as jnp
from jax import lax
from jax.experimental import pallas as pl
from jax.experimental.pallas import tpu as pltpu
from jax.experimental.pallas import tpu_sc as plsc

R = 128
V = 100000
NC, NS = 2, 16
NW = NC * NS
RPW = R // NW
C = 10000
NCHUNK = V // C
GROUPS = C // 16
UF = 5
GP = GROUPS // UF

_E_CACHE = None


def _noise_table():
    global _E_CACHE
    if _E_CACHE is None:
        with jax.ensure_compile_time_eval():
            ekey = jax.random.key(42)
            e = jax.random.exponential(ekey, (R, V), dtype=jnp.float32)
            _E_CACHE = jnp.log(jnp.clip(e, 1e-10, None))
    return _E_CACHE


DEPTH = 4
SLABS = [(r, k) for r in range(RPW) for k in range(NCHUNK)]


def _body(logits_hbm, e_hbm, ts_hbm, em_hbm, out_hbm,
          lbufs, ebufs, tbuf, embuf, obuf, sls, ses):
    cid = lax.axis_index("c")
    sid = lax.axis_index("s")
    wid = cid * NS + sid
    base = wid * RPW

    pltpu.sync_copy(ts_hbm.at[pl.ds(base, RPW)], tbuf)
    pltpu.sync_copy(em_hbm.at[pl.ds(base, RPW)], embuf)

    iota = lax.iota(jnp.int32, 16)
    big = jnp.full((16,), jnp.int32(2147483647), jnp.int32)
    ovec = jnp.zeros((16,), jnp.int32)

    def start(s):
        r, k = SLABS[s]
        b = s % DEPTH
        row = base + r
        pltpu.async_copy(logits_hbm.at[row, pl.ds(k * C, C)], lbufs[b], sls[b])
        pltpu.async_copy(e_hbm.at[row, pl.ds(k * C, C)], ebufs[b], ses[b])

    def wait(s):
        r, k = SLABS[s]
        b = s % DEPTH
        row = base + r
        pltpu.make_async_copy(
            logits_hbm.at[row, pl.ds(k * C, C)], lbufs[b], sls[b]).wait()
        pltpu.make_async_copy(
            e_hbm.at[row, pl.ds(k * C, C)], ebufs[b], ses[b]).wait()

    def compute_chunk(k, lref, eref, acc, tv, emv):
        def it(j, acc):
            new = []
            for u in range(UF):
                vm, vi = acc[2 * u], acc[2 * u + 1]
                off = (j * UF + u) * 16
                l = lref[pl.ds(off, 16)]
                e = eref[pl.ds(off, 16)]
                v = l / tv - emv * e
                g = k * GROUPS + j * UF + u
                cur = jnp.full((16,), g, jnp.int32)
                m = v > vm
                new.append(jnp.where(m, v, vm))
                new.append(jnp.where(m, cur, vi))
            return tuple(new)
        return lax.fori_loop(0, GP, it, acc)

    for s in range(DEPTH - 1):
        start(s)

    acc = None
    for s in range(len(SLABS)):
        r, k = SLABS[s]
        if k == 0:
            acc0 = []
            for u in range(UF):
                acc0.append(jnp.full((16,), -jnp.inf, jnp.float32))
                acc0.append(jnp.zeros((16,), jnp.int32))
            acc = tuple(acc0)
        wait(s)
        if s + DEPTH - 1 < len(SLABS):
            start(s + DEPTH - 1)
        acc = compute_chunk(k, lbufs[s % DEPTH], ebufs[s % DEPTH], acc,
                            tbuf[r], embuf[r])
        if k == NCHUNK - 1:
            vm, vi = acc[0], acc[1]
            for u in range(1, UF):
                vmu, viu = acc[2 * u], acc[2 * u + 1]
                better = (vmu > vm) | ((vmu == vm) & (viu < vi))
                vm = jnp.where(better, vmu, vm)
                vi = jnp.where(better, viu, vi)
            m_all = jnp.max(vm)
            cand = jnp.where(vm == m_all, vi * 16 + iota, big)
            best = jnp.min(cand)
            ovec = jnp.where(iota == r, best, ovec)

    obuf[...] = ovec
    pltpu.sync_copy(obuf, out_hbm.at[wid])


@jax.jit
def _sample(logits, temps, e_tab):
    greedy = temps == 0.0
    ts = jnp.where(greedy, 1.0, temps).astype(jnp.float32)
    em = jnp.where(greedy, 0.0, 1.0).astype(jnp.float32)
    ts_b = jnp.broadcast_to(ts[:, None], (R, 16))
    em_b = jnp.broadcast_to(em[:, None], (R, 16))

    mesh = plsc.VectorSubcoreMesh(
        core_axis_name="c", subcore_axis_name="s", num_cores=NC, num_subcores=NS
    )
    run = pl.kernel(
        _body,
        out_type=jax.ShapeDtypeStruct((NW, 16), jnp.int32),
        mesh=mesh,
        compiler_params=pltpu.CompilerParams(
            use_tc_tiling_on_sc=False, needs_layout_passes=False
        ),
        scratch_types=[
            [pltpu.VMEM((C,), jnp.float32) for _ in range(DEPTH)],
            [pltpu.VMEM((C,), jnp.float32) for _ in range(DEPTH)],
            pltpu.VMEM((RPW, 16), jnp.float32),
            pltpu.VMEM((RPW, 16), jnp.float32),
            pltpu.VMEM((16,), jnp.int32),
            [pltpu.SemaphoreType.DMA for _ in range(DEPTH)],
            [pltpu.SemaphoreType.DMA for _ in range(DEPTH)],
        ],
    )
    res = run(logits, e_tab, ts_b, em_b)
    return res[:, :RPW].reshape(-1)


def kernel(logits, temperatures):
    e_tab = _noise_table()
    temps = temperatures.reshape(-1).astype(jnp.float32)
    return _sample(logits.astype(jnp.float32), temps, e_tab)
